# Initial kernel scaffold; baseline (speedup 1.0000x reference)
import jax
import jax.numpy as jnp
from jax.experimental import pallas as pl


def kernel(x, edge_index, W1, a_s1, a_d1, b1, W2, a_s2, a_d2, b2):
    raise NotImplementedError("write your pallas kernel here")


# speedup vs baseline: 29.2861x; 29.2861x over previous
import functools

import jax
import jax.numpy as jnp
from jax import lax
from jax.experimental import pallas as pl
from jax.experimental.pallas import tpu as pltpu
from jax.experimental.pallas import tpu_sc as plsc

N = 10000
E = 320000
D = 128
C = 128

NC = 2
NS = 16
NW = NC * NS
NPAD = 10240
EW = E // NW
K = 80
NCHUNK = EW // K
GK = K // 16
RPT = NPAD // NS


def _mm_logits_body(x_ref, w_ref, vs_ref, vd_ref, h_ref, as_ref, ad_ref):
    xb = x_ref[...]
    h_ref[...] = jnp.dot(xb, w_ref[...], preferred_element_type=jnp.float32)
    as_ref[...] = jnp.sum(xb * vs_ref[...], axis=1)
    ad_ref[...] = jnp.sum(xb * vd_ref[...], axis=1)


def _mm_logits(x, w, vs, vd):
    return pl.pallas_call(
        _mm_logits_body,
        out_shape=(
            jax.ShapeDtypeStruct((N, C), jnp.float32),
            jax.ShapeDtypeStruct((N,), jnp.float32),
            jax.ShapeDtypeStruct((N,), jnp.float32),
        ),
    )(x, w, vs, vd)


def _norm_mm_logits_body(acc_a_ref, acc_b_ref, den_a_ref, den_b_ref, b_ref,
                         w_ref, vs_ref, vd_ref, h_ref, as_ref, ad_ref):
    den = den_a_ref[...] + den_b_ref[...] + 1e-16
    acc = acc_a_ref[...] + acc_b_ref[...]
    x2 = acc / den[:, None] + b_ref[...]
    x2 = jnp.where(x2 > 0, x2, jnp.exp(x2) - 1.0)
    h_ref[...] = jnp.dot(x2, w_ref[...], preferred_element_type=jnp.float32)
    as_ref[...] = jnp.sum(x2 * vs_ref[...], axis=1)
    ad_ref[...] = jnp.sum(x2 * vd_ref[...], axis=1)


def _norm_mm_logits(acc_a, acc_b, den_a, den_b, b, w, vs, vd):
    return pl.pallas_call(
        _norm_mm_logits_body,
        out_shape=(
            jax.ShapeDtypeStruct((N, C), jnp.float32),
            jax.ShapeDtypeStruct((N,), jnp.float32),
            jax.ShapeDtypeStruct((N,), jnp.float32),
        ),
    )(acc_a, acc_b, den_a, den_b, b, w, vs, vd)


def _norm_out_body(acc_a_ref, acc_b_ref, den_a_ref, den_b_ref, b_ref, o_ref):
    den = den_a_ref[...] + den_b_ref[...] + 1e-16
    acc = acc_a_ref[...] + acc_b_ref[...]
    o_ref[...] = acc / den[:, None] + b_ref[...]


def _norm_out(acc_a, acc_b, den_a, den_b, b):
    return pl.pallas_call(
        _norm_out_body,
        out_shape=jax.ShapeDtypeStruct((N, C), jnp.float32),
    )(acc_a, acc_b, den_a, den_b, b)


_MESH = plsc.VectorSubcoreMesh(core_axis_name="c", subcore_axis_name="s")

NB = 25
NBLK = NCHUNK // NB


@functools.partial(
    pl.kernel,
    out_type=jax.ShapeDtypeStruct((NW, NBLK, NB, K), jnp.float32),
    mesh=_MESH,
    compiler_params=pltpu.CompilerParams(needs_layout_passes=False, use_tc_tiling_on_sc=False),
    scratch_types=(
        pltpu.VMEM((N,), jnp.float32),
        pltpu.VMEM((N,), jnp.float32),
        pltpu.VMEM((NBLK, NB, K), jnp.int32),
        pltpu.VMEM((NBLK, NB, K), jnp.int32),
        pltpu.VMEM((NBLK, NB, K), jnp.float32),
    ),
)
def _sc_logits(asv_hbm, adv_hbm, src_hbm, dst_hbm, ex_out,
               as_t, ad_t, src_t, dst_t, ex_t):
    c = lax.axis_index("c")
    s = lax.axis_index("s")
    w = s * NC + c

    pltpu.sync_copy(asv_hbm, as_t)
    pltpu.sync_copy(adv_hbm, ad_t)
    pltpu.sync_copy(src_hbm.at[w], src_t)
    pltpu.sync_copy(dst_hbm.at[w], dst_t)

    def block(ib, carry):
        def chunk(j, carry2):
            for g in range(GK):
                sl = pl.ds(g * 16, 16)
                av = plsc.load_gather(as_t, [src_t[ib, j, sl]])
                bv = plsc.load_gather(ad_t, [dst_t[ib, j, sl]])
                e = av + bv
                e = jnp.where(e > 0, e, 0.2 * e)
                ex_t[ib, j, sl] = jnp.exp(e)
            return carry2
        lax.fori_loop(0, NB, chunk, 0)
        return carry

    lax.fori_loop(0, NBLK, block, 0)
    pltpu.sync_copy(ex_t, ex_out.at[w])


@functools.partial(
    pl.kernel,
    out_type=(
        jax.ShapeDtypeStruct((NC, NPAD, C), jnp.float32),
        jax.ShapeDtypeStruct((NC, NPAD), jnp.float32),
    ),
    mesh=_MESH,
    compiler_params=pltpu.CompilerParams(needs_layout_passes=False, use_tc_tiling_on_sc=False),
    scratch_types=(
        pltpu.VMEM((NB, K), jnp.int32),
        pltpu.VMEM((NB, K), jnp.int32),
        pltpu.VMEM((NB, K), jnp.float32),
        pltpu.VMEM((K, C), jnp.float32),
        pltpu.VMEM_SHARED((NPAD, C), jnp.float32),
        pltpu.VMEM_SHARED((NPAD,), jnp.float32),
    ),
)
def _sc_edge(h_hbm, src_hbm, dst_hbm, ex_hbm, z2_hbm, z1_hbm,
             acc_out, den_out,
             src_b, dst_b, ex_b, rows_t, acc_sh, den_sh):
    c = lax.axis_index("c")
    s = lax.axis_index("s")
    w = s * NC + c

    rb = s * RPT
    pltpu.sync_copy(z2_hbm.at[pl.ds(rb, RPT)], acc_sh.at[pl.ds(rb, RPT)])
    pltpu.sync_copy(z1_hbm.at[pl.ds(rb, RPT)], den_sh.at[pl.ds(rb, RPT)])
    plsc.subcore_barrier()

    def block(ib, carry):
        pltpu.sync_copy(src_hbm.at[w, ib], src_b)
        pltpu.sync_copy(dst_hbm.at[w, ib], dst_b)
        pltpu.sync_copy(ex_hbm.at[w, ib], ex_b)

        def chunk(j, carry2):
            pltpu.sync_copy(h_hbm.at[src_b.at[j]], rows_t)
            for g in range(GK):
                ev = ex_b[j, pl.ds(g * 16, 16)]
                for l in range(16):
                    exs = ev[l]
                    k = g * 16 + l
                    for jj in range(C // 16):
                        cs = pl.ds(jj * 16, 16)
                        rows_t[k, cs] = rows_t[k, cs] * exs
            pltpu.sync_copy(rows_t, acc_sh.at[dst_b.at[j]], add=True)
            pltpu.sync_copy(ex_b.at[j], den_sh.at[dst_b.at[j]], add=True)
            return carry2

        lax.fori_loop(0, NB, chunk, 0)
        return carry

    lax.fori_loop(0, NBLK, block, 0)

    plsc.subcore_barrier()
    pltpu.sync_copy(acc_sh.at[pl.ds(rb, RPT)], acc_out.at[c, pl.ds(rb, RPT)])
    pltpu.sync_copy(den_sh.at[pl.ds(rb, RPT)], den_out.at[c, pl.ds(rb, RPT)])


def kernel(x, edge_index, W1, a_s1, a_d1, b1, W2, a_s2, a_d2, b2):
    src = edge_index[0].reshape(NW, NBLK, NB, K)
    dst = edge_index[1].reshape(NW, NBLK, NB, K)
    z2 = jnp.zeros((NPAD, C), jnp.float32)
    z1 = jnp.zeros((NPAD,), jnp.float32)

    vs1 = W1 @ a_s1.reshape(C)
    vd1 = W1 @ a_d1.reshape(C)
    vs2 = W2 @ a_s2.reshape(C)
    vd2 = W2 @ a_d2.reshape(C)

    h1, as1, ad1 = _mm_logits(x, W1, vs1.reshape(1, C), vd1.reshape(1, C))
    ex1 = _sc_logits(as1, ad1, src, dst)
    acc1, den1 = _sc_edge(h1, src, dst, ex1, z2, z1)
    h2, as2, ad2 = _norm_mm_logits(
        acc1[0, :N], acc1[1, :N], den1[0, :N], den1[1, :N],
        b1.reshape(1, C), W2, vs2.reshape(1, C), vd2.reshape(1, C))
    ex2 = _sc_logits(as2, ad2, src, dst)
    acc2, den2 = _sc_edge(h2, src, dst, ex2, z2, z1)
    out = _norm_out(acc2[0, :N], acc2[1, :N], den2[0, :N], den2[1, :N],
                    b2.reshape(1, C))
    return out


# speedup vs baseline: 39.6471x; 1.3538x over previous
import functools

import jax
import jax.numpy as jnp
from jax import lax
from jax.experimental import pallas as pl
from jax.experimental.pallas import tpu as pltpu
from jax.experimental.pallas import tpu_sc as plsc

N = 10000
E = 320000
D = 128
C = 128

NC = 2
NS = 16
NW = NC * NS
NPAD = 10240
EW = E // NW
K = 80
NCHUNK = EW // K
GK = K // 16
RPT = NPAD // NS


def _mm_logits_body(x_ref, w_ref, vs_ref, vd_ref, h_ref, as_ref, ad_ref):
    xb = x_ref[...]
    h_ref[...] = jnp.dot(xb, w_ref[...], preferred_element_type=jnp.float32)
    as_ref[...] = jnp.sum(xb * vs_ref[...], axis=1)
    ad_ref[...] = jnp.sum(xb * vd_ref[...], axis=1)


def _mm_logits(x, w, vs, vd):
    return pl.pallas_call(
        _mm_logits_body,
        out_shape=(
            jax.ShapeDtypeStruct((N, C), jnp.float32),
            jax.ShapeDtypeStruct((N,), jnp.float32),
            jax.ShapeDtypeStruct((N,), jnp.float32),
        ),
    )(x, w, vs, vd)


def _norm_mm_logits_body(acc_a_ref, acc_b_ref, den_a_ref, den_b_ref, b_ref,
                         w_ref, vs_ref, vd_ref, h_ref, as_ref, ad_ref):
    den = den_a_ref[...] + den_b_ref[...] + 1e-16
    acc = acc_a_ref[...] + acc_b_ref[...]
    x2 = acc / den[:, None] + b_ref[...]
    x2 = jnp.where(x2 > 0, x2, jnp.exp(x2) - 1.0)
    h_ref[...] = jnp.dot(x2, w_ref[...], preferred_element_type=jnp.float32)
    as_ref[...] = jnp.sum(x2 * vs_ref[...], axis=1)
    ad_ref[...] = jnp.sum(x2 * vd_ref[...], axis=1)


def _norm_mm_logits(acc_a, acc_b, den_a, den_b, b, w, vs, vd):
    return pl.pallas_call(
        _norm_mm_logits_body,
        out_shape=(
            jax.ShapeDtypeStruct((N, C), jnp.float32),
            jax.ShapeDtypeStruct((N,), jnp.float32),
            jax.ShapeDtypeStruct((N,), jnp.float32),
        ),
    )(acc_a, acc_b, den_a, den_b, b, w, vs, vd)


def _norm_out_body(acc_a_ref, acc_b_ref, den_a_ref, den_b_ref, b_ref, o_ref):
    den = den_a_ref[...] + den_b_ref[...] + 1e-16
    acc = acc_a_ref[...] + acc_b_ref[...]
    o_ref[...] = acc / den[:, None] + b_ref[...]


def _norm_out(acc_a, acc_b, den_a, den_b, b):
    return pl.pallas_call(
        _norm_out_body,
        out_shape=jax.ShapeDtypeStruct((N, C), jnp.float32),
    )(acc_a, acc_b, den_a, den_b, b)


_MESH = plsc.VectorSubcoreMesh(core_axis_name="c", subcore_axis_name="s")

NB = 5
NBLK = NCHUNK // NB


@functools.partial(
    pl.kernel,
    out_type=jax.ShapeDtypeStruct((NW, NBLK, NB, K), jnp.float32),
    mesh=_MESH,
    compiler_params=pltpu.CompilerParams(needs_layout_passes=False, use_tc_tiling_on_sc=False),
    scratch_types=(
        pltpu.VMEM((N,), jnp.float32),
        pltpu.VMEM((N,), jnp.float32),
        pltpu.VMEM((NBLK, NB, K), jnp.int32),
        pltpu.VMEM((NBLK, NB, K), jnp.int32),
        pltpu.VMEM((NBLK, NB, K), jnp.float32),
    ),
)
def _sc_logits(asv_hbm, adv_hbm, src_hbm, dst_hbm, ex_out,
               as_t, ad_t, src_t, dst_t, ex_t):
    c = lax.axis_index("c")
    s = lax.axis_index("s")
    w = s * NC + c

    pltpu.sync_copy(asv_hbm, as_t)
    pltpu.sync_copy(adv_hbm, ad_t)
    pltpu.sync_copy(src_hbm.at[w], src_t)
    pltpu.sync_copy(dst_hbm.at[w], dst_t)

    def block(ib, carry):
        def chunk(j, carry2):
            for g in range(GK):
                sl = pl.ds(g * 16, 16)
                av = plsc.load_gather(as_t, [src_t[ib, j, sl]])
                bv = plsc.load_gather(ad_t, [dst_t[ib, j, sl]])
                e = av + bv
                e = jnp.where(e > 0, e, 0.2 * e)
                ex_t[ib, j, sl] = jnp.exp(e)
            return carry2
        lax.fori_loop(0, NB, chunk, 0)
        return carry

    lax.fori_loop(0, NBLK, block, 0)
    pltpu.sync_copy(ex_t, ex_out.at[w])


@functools.partial(
    pl.kernel,
    out_type=(
        jax.ShapeDtypeStruct((NC, NPAD, C), jnp.float32),
        jax.ShapeDtypeStruct((NC, NPAD), jnp.float32),
    ),
    mesh=_MESH,
    compiler_params=pltpu.CompilerParams(needs_layout_passes=False, use_tc_tiling_on_sc=False),
    scratch_types=(
        pltpu.VMEM((2, NB, K), jnp.int32),
        pltpu.VMEM((2, NB, K), jnp.int32),
        pltpu.VMEM((2, NB, K), jnp.float32),
        pltpu.VMEM((K, C), jnp.float32),
        pltpu.VMEM((K, C), jnp.float32),
        pltpu.VMEM((K, C), jnp.float32),
        pltpu.VMEM_SHARED((NPAD, C), jnp.float32),
        pltpu.VMEM_SHARED((NPAD,), jnp.float32),
        pltpu.SemaphoreType.DMA,
        pltpu.SemaphoreType.DMA,
        pltpu.SemaphoreType.DMA,
        pltpu.SemaphoreType.DMA,
        pltpu.SemaphoreType.DMA,
        pltpu.SemaphoreType.DMA,
        pltpu.SemaphoreType.DMA,
    ),
)
def _sc_edge(h_hbm, src_hbm, dst_hbm, ex_hbm, z2_hbm, z1_hbm,
             acc_out, den_out,
             src_b, dst_b, ex_b, rows0, rows1, rows2, acc_sh, den_sh,
             gs0, gs1, gs2, ss0, ss1, ss2, stg):
    c = lax.axis_index("c")
    s = lax.axis_index("s")
    w = s * NC + c
    rows = (rows0, rows1, rows2)
    gsem = (gs0, gs1, gs2)
    ssem = (ss0, ss1, ss2)
    last = NCHUNK - 1

    rb = s * RPT
    pltpu.sync_copy(z2_hbm.at[pl.ds(rb, RPT)], acc_sh.at[pl.ds(rb, RPT)])
    pltpu.sync_copy(z1_hbm.at[pl.ds(rb, RPT)], den_sh.at[pl.ds(rb, RPT)])
    plsc.subcore_barrier()

    def slot_row(t):
        return (t // NB) % 2, t % NB

    def start_gather(t, u):
        sl, r = slot_row(t)
        pltpu.async_copy(h_hbm.at[src_b.at[sl, r]], rows[u], gsem[u])

    def do_chunk(i, u):
        sl, r = slot_row(i)
        up1 = (u + 1) % 3
        up2 = (u + 2) % 3
        pltpu.make_async_copy(h_hbm.at[src_b.at[sl, r]], rows[u], gsem[u]).wait()
        bi = i % NB
        ib = i // NB

        @pl.when(jnp.logical_and(bi == 0, ib < NBLK - 1))
        def _():
            nsl = (ib + 1) % 2
            pltpu.async_copy(src_hbm.at[w, ib + 1], src_b.at[nsl], stg)
            pltpu.async_copy(dst_hbm.at[w, ib + 1], dst_b.at[nsl], stg)
            pltpu.async_copy(ex_hbm.at[w, ib + 1], ex_b.at[nsl], stg)

        for g in range(GK):
            ev = ex_b[sl, r, pl.ds(g * 16, 16)]
            for l in range(16):
                exs = ev[l]
                k = g * 16 + l
                for jj in range(C // 16):
                    cs = pl.ds(jj * 16, 16)
                    rows[u][k, cs] = rows[u][k, cs] * exs
        pltpu.async_copy(rows[u], acc_sh.at[dst_b.at[sl, r]], ssem[u], add=True)
        pltpu.sync_copy(ex_b.at[sl, r], den_sh.at[dst_b.at[sl, r]], add=True)

        @pl.when(jnp.logical_and(bi == NB - 2, ib < NBLK - 1))
        def _():
            nsl = (ib + 1) % 2
            pltpu.make_async_copy(src_hbm.at[w, ib + 1], src_b.at[nsl], stg).wait()
            pltpu.make_async_copy(dst_hbm.at[w, ib + 1], dst_b.at[nsl], stg).wait()
            pltpu.make_async_copy(ex_hbm.at[w, ib + 1], ex_b.at[nsl], stg).wait()

        @pl.when(i >= 1)
        def _():
            psl, pr = slot_row(i - 1)
            pltpu.make_async_copy(
                rows[up2], acc_sh.at[dst_b.at[psl, pr]], ssem[up2]).wait()

        @pl.when(i + 2 <= last)
        def _():
            start_gather(i + 2, up2)

    pltpu.sync_copy(src_hbm.at[w, 0], src_b.at[0])
    pltpu.sync_copy(dst_hbm.at[w, 0], dst_b.at[0])
    pltpu.sync_copy(ex_hbm.at[w, 0], ex_b.at[0])
    start_gather(0, 0)
    start_gather(1, 1)

    def group(g3, carry):
        i0 = g3 * 3
        do_chunk(i0, 0)
        do_chunk(i0 + 1, 1)
        do_chunk(i0 + 2, 2)
        return carry

    ngroups = (NCHUNK - 2) // 3
    lax.fori_loop(0, ngroups, group, 0)
    do_chunk(jnp.int32(NCHUNK - 2), 0)
    do_chunk(jnp.int32(NCHUNK - 1), 1)
    fsl, fr = slot_row(last)
    pltpu.make_async_copy(rows[1], acc_sh.at[dst_b.at[fsl, fr]], ssem[1]).wait()

    plsc.subcore_barrier()
    pltpu.sync_copy(acc_sh.at[pl.ds(rb, RPT)], acc_out.at[c, pl.ds(rb, RPT)])
    pltpu.sync_copy(den_sh.at[pl.ds(rb, RPT)], den_out.at[c, pl.ds(rb, RPT)])


def kernel(x, edge_index, W1, a_s1, a_d1, b1, W2, a_s2, a_d2, b2):
    src = edge_index[0].reshape(NW, NBLK, NB, K)
    dst = edge_index[1].reshape(NW, NBLK, NB, K)
    z2 = jnp.zeros((NPAD, C), jnp.float32)
    z1 = jnp.zeros((NPAD,), jnp.float32)

    vs1 = W1 @ a_s1.reshape(C)
    vd1 = W1 @ a_d1.reshape(C)
    vs2 = W2 @ a_s2.reshape(C)
    vd2 = W2 @ a_d2.reshape(C)

    h1, as1, ad1 = _mm_logits(x, W1, vs1.reshape(1, C), vd1.reshape(1, C))
    ex1 = _sc_logits(as1, ad1, src, dst)
    acc1, den1 = _sc_edge(h1, src, dst, ex1, z2, z1)
    h2, as2, ad2 = _norm_mm_logits(
        acc1[0, :N], acc1[1, :N], den1[0, :N], den1[1, :N],
        b1.reshape(1, C), W2, vs2.reshape(1, C), vd2.reshape(1, C))
    ex2 = _sc_logits(as2, ad2, src, dst)
    acc2, den2 = _sc_edge(h2, src, dst, ex2, z2, z1)
    out = _norm_out(acc2[0, :N], acc2[1, :N], den2[0, :N], den2[1, :N],
                    b2.reshape(1, C))
    return out


# speedup vs baseline: 54.7401x; 1.3807x over previous
import functools

import jax
import jax.numpy as jnp
from jax import lax
from jax.experimental import pallas as pl
from jax.experimental.pallas import tpu as pltpu
from jax.experimental.pallas import tpu_sc as plsc

N = 10000
E = 320000
D = 128
C = 128

NC = 2
NS = 16
NW = NC * NS
NPAD = 10240
EW = E // NW
K = 80
NCHUNK = EW // K
GK = K // 16
RPT = NPAD // NS


def _mm_logits_body(x_ref, w_ref, vs_ref, vd_ref, h_ref, as_ref, ad_ref):
    xb = x_ref[...]
    h_ref[...] = jnp.dot(xb, w_ref[...], preferred_element_type=jnp.float32)
    as_ref[...] = jnp.sum(xb * vs_ref[...], axis=1)
    ad_ref[...] = jnp.sum(xb * vd_ref[...], axis=1)


def _mm_logits(x, w, vs, vd):
    return pl.pallas_call(
        _mm_logits_body,
        out_shape=(
            jax.ShapeDtypeStruct((N, C), jnp.float32),
            jax.ShapeDtypeStruct((N,), jnp.float32),
            jax.ShapeDtypeStruct((N,), jnp.float32),
        ),
    )(x, w, vs, vd)


def _norm_mm_logits_body(acc_ref, den_ref, b_ref, w_ref, vs_ref, vd_ref,
                         h_ref, as_ref, ad_ref):
    den = den_ref[0] + den_ref[1] + 1e-16
    acc = acc_ref[0] + acc_ref[1]
    x2 = acc / den[:, None] + b_ref[...]
    x2 = jnp.where(x2 > 0, x2, jnp.exp(x2) - 1.0)[:N]
    h_ref[...] = jnp.dot(x2, w_ref[...], preferred_element_type=jnp.float32)
    as_ref[...] = jnp.sum(x2 * vs_ref[...], axis=1)
    ad_ref[...] = jnp.sum(x2 * vd_ref[...], axis=1)


def _norm_mm_logits(acc, den, b, w, vs, vd):
    return pl.pallas_call(
        _norm_mm_logits_body,
        out_shape=(
            jax.ShapeDtypeStruct((N, C), jnp.float32),
            jax.ShapeDtypeStruct((N,), jnp.float32),
            jax.ShapeDtypeStruct((N,), jnp.float32),
        ),
    )(acc, den, b, w, vs, vd)


def _norm_out_body(acc_ref, den_ref, b_ref, o_ref):
    den = den_ref[0] + den_ref[1] + 1e-16
    acc = acc_ref[0] + acc_ref[1]
    o_ref[...] = (acc / den[:, None])[:N] + b_ref[...]


def _norm_out(acc, den, b):
    return pl.pallas_call(
        _norm_out_body,
        out_shape=jax.ShapeDtypeStruct((N, C), jnp.float32),
    )(acc, den, b)


_MESH = plsc.VectorSubcoreMesh(core_axis_name="c", subcore_axis_name="s")

NB = 5
NBLK = NCHUNK // NB


@functools.partial(
    pl.kernel,
    out_type=(
        jax.ShapeDtypeStruct((NC, NPAD, C), jnp.float32),
        jax.ShapeDtypeStruct((NC, NPAD), jnp.float32),
    ),
    mesh=_MESH,
    compiler_params=pltpu.CompilerParams(needs_layout_passes=False, use_tc_tiling_on_sc=False),
    scratch_types=(
        pltpu.VMEM((2, NB, K), jnp.int32),
        pltpu.VMEM((2, NB, K), jnp.int32),
        pltpu.VMEM((K, C), jnp.float32),
        pltpu.VMEM((K, C), jnp.float32),
        pltpu.VMEM((K, C), jnp.float32),
        pltpu.VMEM((3, K), jnp.float32),
        pltpu.VMEM((3, K), jnp.float32),
        pltpu.VMEM((3, K), jnp.float32),
        pltpu.VMEM_SHARED((NPAD, C), jnp.float32),
        pltpu.VMEM_SHARED((NPAD,), jnp.float32),
        pltpu.SemaphoreType.DMA,
        pltpu.SemaphoreType.DMA,
        pltpu.SemaphoreType.DMA,
        pltpu.SemaphoreType.DMA,
        pltpu.SemaphoreType.DMA,
        pltpu.SemaphoreType.DMA,
        pltpu.SemaphoreType.DMA,
    ),
)
def _sc_edge(h_hbm, asv_hbm, adv_hbm, src_hbm, dst_hbm, z2_hbm, z1_hbm,
             acc_out, den_out,
             src_b, dst_b, rows0, rows1, rows2, asx, adx, exv, acc_sh, den_sh,
             gs0, gs1, gs2, ss0, ss1, ss2, stg):
    c = lax.axis_index("c")
    s = lax.axis_index("s")
    w = s * NC + c
    rows = (rows0, rows1, rows2)
    gsem = (gs0, gs1, gs2)
    ssem = (ss0, ss1, ss2)
    last = NCHUNK - 1

    rb = s * RPT
    pltpu.sync_copy(z2_hbm.at[pl.ds(rb, RPT)], acc_sh.at[pl.ds(rb, RPT)])
    pltpu.sync_copy(z1_hbm.at[pl.ds(rb, RPT)], den_sh.at[pl.ds(rb, RPT)])
    plsc.subcore_barrier()

    def slot_row(t):
        return (t // NB) % 2, t % NB

    def start_gather(t, u):
        sl, r = slot_row(t)
        pltpu.async_copy(h_hbm.at[src_b.at[sl, r]], rows[u], gsem[u])
        pltpu.async_copy(asv_hbm.at[src_b.at[sl, r]], asx.at[u], gsem[u])
        pltpu.async_copy(adv_hbm.at[dst_b.at[sl, r]], adx.at[u], gsem[u])

    def wait_gather(i, u):
        sl, r = slot_row(i)
        pltpu.make_async_copy(h_hbm.at[src_b.at[sl, r]], rows[u], gsem[u]).wait()
        pltpu.make_async_copy(asv_hbm.at[src_b.at[sl, r]], asx.at[u], gsem[u]).wait()
        pltpu.make_async_copy(adv_hbm.at[dst_b.at[sl, r]], adx.at[u], gsem[u]).wait()

    def do_chunk(i, u):
        sl, r = slot_row(i)
        up2 = (u + 2) % 3
        wait_gather(i, u)
        bi = i % NB
        ib = i // NB

        @pl.when(jnp.logical_and(bi == 0, ib < NBLK - 1))
        def _():
            nsl = (ib + 1) % 2
            pltpu.async_copy(src_hbm.at[w, ib + 1], src_b.at[nsl], stg)
            pltpu.async_copy(dst_hbm.at[w, ib + 1], dst_b.at[nsl], stg)

        def scale_group(g, carry3):
            gsl = pl.ds(g * 16, 16)
            av = asx[u, gsl]
            bv = adx[u, gsl]
            e = av + bv
            e = jnp.where(e > 0, e, 0.2 * e)
            ev = jnp.exp(e)
            exv[u, gsl] = ev
            for l in range(16):
                exs = ev[l]
                k = g * 16 + l
                for jj in range(C // 16):
                    cs = pl.ds(jj * 16, 16)
                    rows[u][k, cs] = rows[u][k, cs] * exs
            return carry3
        lax.fori_loop(0, GK, scale_group, 0)

        pltpu.async_copy(rows[u], acc_sh.at[dst_b.at[sl, r]], ssem[u], add=True)
        pltpu.sync_copy(exv.at[u], den_sh.at[dst_b.at[sl, r]], add=True)

        @pl.when(jnp.logical_and(bi == NB - 2, ib < NBLK - 1))
        def _():
            nsl = (ib + 1) % 2
            pltpu.make_async_copy(src_hbm.at[w, ib + 1], src_b.at[nsl], stg).wait()
            pltpu.make_async_copy(dst_hbm.at[w, ib + 1], dst_b.at[nsl], stg).wait()

        @pl.when(i >= 1)
        def _():
            psl, pr = slot_row(i - 1)
            pltpu.make_async_copy(
                rows[up2], acc_sh.at[dst_b.at[psl, pr]], ssem[up2]).wait()

        @pl.when(i + 2 <= last)
        def _():
            start_gather(i + 2, up2)

    pltpu.sync_copy(src_hbm.at[w, 0], src_b.at[0])
    pltpu.sync_copy(dst_hbm.at[w, 0], dst_b.at[0])
    start_gather(0, 0)
    start_gather(1, 1)

    def group(g3, carry):
        i0 = g3 * 3
        do_chunk(i0, 0)
        do_chunk(i0 + 1, 1)
        do_chunk(i0 + 2, 2)
        return carry

    ngroups = (NCHUNK - 2) // 3
    lax.fori_loop(0, ngroups, group, 0)
    do_chunk(jnp.int32(NCHUNK - 2), 0)
    do_chunk(jnp.int32(NCHUNK - 1), 1)
    fsl, fr = slot_row(last)
    pltpu.make_async_copy(rows[1], acc_sh.at[dst_b.at[fsl, fr]], ssem[1]).wait()

    plsc.subcore_barrier()
    pltpu.sync_copy(acc_sh.at[pl.ds(rb, RPT)], acc_out.at[c, pl.ds(rb, RPT)])
    pltpu.sync_copy(den_sh.at[pl.ds(rb, RPT)], den_out.at[c, pl.ds(rb, RPT)])


def kernel(x, edge_index, W1, a_s1, a_d1, b1, W2, a_s2, a_d2, b2):
    src = edge_index[0].reshape(NW, NBLK, NB, K)
    dst = edge_index[1].reshape(NW, NBLK, NB, K)
    z2 = jnp.zeros((NPAD, C), jnp.float32)
    z1 = jnp.zeros((NPAD,), jnp.float32)

    vs1 = W1 @ a_s1.reshape(C)
    vd1 = W1 @ a_d1.reshape(C)
    vs2 = W2 @ a_s2.reshape(C)
    vd2 = W2 @ a_d2.reshape(C)

    h1, as1, ad1 = _mm_logits(x, W1, vs1.reshape(1, C), vd1.reshape(1, C))
    acc1, den1 = _sc_edge(h1, as1, ad1, src, dst, z2, z1)
    h2, as2, ad2 = _norm_mm_logits(
        acc1, den1, b1.reshape(1, C), W2, vs2.reshape(1, C), vd2.reshape(1, C))
    acc2, den2 = _sc_edge(h2, as2, ad2, src, dst, z2, z1)
    out = _norm_out(acc2, den2, b2.reshape(1, C))
    return out


# speedup vs baseline: 56.1592x; 1.0259x over previous
import functools

import jax
import jax.numpy as jnp
from jax import lax
from jax.experimental import pallas as pl
from jax.experimental.pallas import tpu as pltpu
from jax.experimental.pallas import tpu_sc as plsc

N = 10000
E = 320000
D = 128
C = 128

NC = 2
NS = 16
NW = NC * NS
NPAD = 10240
EW = E // NW
K = 80
NCHUNK = EW // K
GK = K // 16
RPT = NPAD // NS


def _mm_logits_body(x_ref, w_ref, vs_ref, vd_ref, h_ref, as_ref, ad_ref):
    xb = x_ref[...]
    h_ref[...] = jnp.dot(xb, w_ref[...], preferred_element_type=jnp.float32)
    as_ref[...] = jnp.sum(xb * vs_ref[...], axis=1)
    ad_ref[...] = jnp.sum(xb * vd_ref[...], axis=1)


def _mm_logits(x, w, vs, vd):
    return pl.pallas_call(
        _mm_logits_body,
        out_shape=(
            jax.ShapeDtypeStruct((N, C), jnp.float32),
            jax.ShapeDtypeStruct((N,), jnp.float32),
            jax.ShapeDtypeStruct((N,), jnp.float32),
        ),
    )(x, w, vs, vd)


def _norm_mm_logits_body(acc_ref, den_ref, b_ref, w_ref, vs_ref, vd_ref,
                         h_ref, as_ref, ad_ref):
    den = den_ref[0] + den_ref[1] + 1e-16
    acc = acc_ref[0] + acc_ref[1]
    x2 = acc / den[:, None] + b_ref[...]
    x2 = jnp.where(x2 > 0, x2, jnp.exp(x2) - 1.0)[:N]
    h_ref[...] = jnp.dot(x2, w_ref[...], preferred_element_type=jnp.float32)
    as_ref[...] = jnp.sum(x2 * vs_ref[...], axis=1)
    ad_ref[...] = jnp.sum(x2 * vd_ref[...], axis=1)


def _norm_mm_logits(acc, den, b, w, vs, vd):
    return pl.pallas_call(
        _norm_mm_logits_body,
        out_shape=(
            jax.ShapeDtypeStruct((N, C), jnp.float32),
            jax.ShapeDtypeStruct((N,), jnp.float32),
            jax.ShapeDtypeStruct((N,), jnp.float32),
        ),
    )(acc, den, b, w, vs, vd)


def _norm_out_body(acc_ref, den_ref, b_ref, o_ref):
    den = den_ref[0] + den_ref[1] + 1e-16
    acc = acc_ref[0] + acc_ref[1]
    o_ref[...] = (acc / den[:, None])[:N] + b_ref[...]


def _norm_out(acc, den, b):
    return pl.pallas_call(
        _norm_out_body,
        out_shape=jax.ShapeDtypeStruct((N, C), jnp.float32),
    )(acc, den, b)


_MESH = plsc.VectorSubcoreMesh(core_axis_name="c", subcore_axis_name="s")

NB = 5
NBLK = NCHUNK // NB


@functools.partial(
    pl.kernel,
    out_type=(
        jax.ShapeDtypeStruct((NC, NPAD, C), jnp.float32),
        jax.ShapeDtypeStruct((NC, NPAD), jnp.float32),
    ),
    mesh=_MESH,
    compiler_params=pltpu.CompilerParams(needs_layout_passes=False, use_tc_tiling_on_sc=False),
    scratch_types=(
        pltpu.VMEM((2, 1, NB * K), jnp.int32),
        pltpu.VMEM((2, 1, NB * K), jnp.int32),
        pltpu.VMEM((K, C), jnp.float32),
        pltpu.VMEM((K, C), jnp.float32),
        pltpu.VMEM((K, C), jnp.float32),
        pltpu.VMEM((2, 1, NB * K), jnp.float32),
        pltpu.VMEM((2, 1, NB * K), jnp.float32),
        pltpu.VMEM((2, 1, NB * K), jnp.float32),
        pltpu.VMEM_SHARED((NPAD, C), jnp.float32),
        pltpu.VMEM_SHARED((NPAD,), jnp.float32),
        pltpu.SemaphoreType.DMA,
        pltpu.SemaphoreType.DMA,
        pltpu.SemaphoreType.DMA,
        pltpu.SemaphoreType.DMA,
        pltpu.SemaphoreType.DMA,
        pltpu.SemaphoreType.DMA,
        pltpu.SemaphoreType.DMA,
        pltpu.SemaphoreType.DMA,
        pltpu.SemaphoreType.DMA,
    ),
)
def _sc_edge(h_hbm, asv_hbm, adv_hbm, src_hbm, dst_hbm, z2_hbm, z1_hbm,
             acc_out, den_out,
             src_b, dst_b, rows0, rows1, rows2, asx, adx, exv, acc_sh, den_sh,
             gs0, gs1, gs2, ss0, ss1, ss2, stg, ags, dsm):
    c = lax.axis_index("c")
    s = lax.axis_index("s")
    w = s * NC + c
    rows = (rows0, rows1, rows2)
    gsem = (gs0, gs1, gs2)
    ssem = (ss0, ss1, ss2)
    last = NCHUNK - 1

    rb = s * RPT
    pltpu.sync_copy(z2_hbm.at[pl.ds(rb, RPT)], acc_sh.at[pl.ds(rb, RPT)])
    pltpu.sync_copy(z1_hbm.at[pl.ds(rb, RPT)], den_sh.at[pl.ds(rb, RPT)])
    plsc.subcore_barrier()

    def slot_row(t):
        return (t // NB) % 2, t % NB

    def start_gather(t, u):
        sl, r = slot_row(t)
        pltpu.async_copy(h_hbm.at[src_b.at[sl, 0, pl.ds(r * K, K)]],
                         rows[u], gsem[u])

    def wait_gather(i, u):
        sl, r = slot_row(i)
        pltpu.make_async_copy(h_hbm.at[src_b.at[sl, 0, pl.ds(r * K, K)]],
                              rows[u], gsem[u]).wait()

    def start_logit_gather(sl2):
        pltpu.async_copy(asv_hbm.at[src_b.at[sl2, 0]], asx.at[sl2, 0], ags)
        pltpu.async_copy(adv_hbm.at[dst_b.at[sl2, 0]], adx.at[sl2, 0], ags)

    def wait_logit_gather(sl2):
        pltpu.make_async_copy(asv_hbm.at[src_b.at[sl2, 0]], asx.at[sl2, 0], ags).wait()
        pltpu.make_async_copy(adv_hbm.at[dst_b.at[sl2, 0]], adx.at[sl2, 0], ags).wait()

    def do_chunk(i, u):
        sl, r = slot_row(i)
        up2 = (u + 2) % 3
        wait_gather(i, u)
        bi = i % NB
        ib = i // NB

        @pl.when(bi == 0)
        def _():
            wait_logit_gather(sl)

        @pl.when(jnp.logical_and(bi == 0, ib >= 2))
        def _():
            pltpu.make_async_copy(
                exv.at[sl, 0], den_sh.at[dst_b.at[sl, 0]], dsm).wait()

        @pl.when(jnp.logical_and(bi == 0, ib < NBLK - 1))
        def _():
            nsl = (ib + 1) % 2
            pltpu.async_copy(src_hbm.at[w, ib + 1], src_b.at[nsl], stg)
            pltpu.async_copy(dst_hbm.at[w, ib + 1], dst_b.at[nsl], stg)

        def scale_group(g, carry3):
            gsl = pl.ds(r * K + g * 16, 16)
            av = asx[sl, 0, gsl]
            bv = adx[sl, 0, gsl]
            e = av + bv
            e = jnp.where(e > 0, e, 0.2 * e)
            ev = jnp.exp(e)
            exv[sl, 0, gsl] = ev
            for l in range(16):
                exs = ev[l]
                k = g * 16 + l
                for jj in range(C // 16):
                    cs = pl.ds(jj * 16, 16)
                    rows[u][k, cs] = rows[u][k, cs] * exs
            return carry3
        lax.fori_loop(0, GK, scale_group, 0)

        pltpu.async_copy(rows[u], acc_sh.at[dst_b.at[sl, 0, pl.ds(r * K, K)]],
                         ssem[u], add=True)

        @pl.when(jnp.logical_and(bi == NB - 2, ib < NBLK - 1))
        def _():
            nsl = (ib + 1) % 2
            pltpu.make_async_copy(src_hbm.at[w, ib + 1], src_b.at[nsl], stg).wait()
            pltpu.make_async_copy(dst_hbm.at[w, ib + 1], dst_b.at[nsl], stg).wait()
            start_logit_gather(nsl)

        @pl.when(bi == NB - 1)
        def _():
            pltpu.async_copy(exv.at[sl, 0], den_sh.at[dst_b.at[sl, 0]], dsm, add=True)

        @pl.when(i >= 1)
        def _():
            psl, pr = slot_row(i - 1)
            pltpu.make_async_copy(
                rows[up2], acc_sh.at[dst_b.at[psl, 0, pl.ds(pr * K, K)]],
                ssem[up2]).wait()

        @pl.when(i + 2 <= last)
        def _():
            start_gather(i + 2, up2)

    pltpu.sync_copy(src_hbm.at[w, 0], src_b.at[0])
    pltpu.sync_copy(dst_hbm.at[w, 0], dst_b.at[0])
    start_logit_gather(0)
    start_gather(0, 0)
    start_gather(1, 1)

    def group(g3, carry):
        i0 = g3 * 3
        do_chunk(i0, 0)
        do_chunk(i0 + 1, 1)
        do_chunk(i0 + 2, 2)
        return carry

    ngroups = (NCHUNK - 2) // 3
    lax.fori_loop(0, ngroups, group, 0)
    do_chunk(jnp.int32(NCHUNK - 2), 0)
    do_chunk(jnp.int32(NCHUNK - 1), 1)
    fsl, fr = slot_row(last)
    pltpu.make_async_copy(rows[1], acc_sh.at[dst_b.at[fsl, 0, pl.ds(fr * K, K)]],
                          ssem[1]).wait()
    pltpu.make_async_copy(exv.at[0, 0], den_sh.at[dst_b.at[0, 0]], dsm).wait()
    pltpu.make_async_copy(exv.at[1, 0], den_sh.at[dst_b.at[1, 0]], dsm).wait()

    plsc.subcore_barrier()
    pltpu.sync_copy(acc_sh.at[pl.ds(rb, RPT)], acc_out.at[c, pl.ds(rb, RPT)])
    pltpu.sync_copy(den_sh.at[pl.ds(rb, RPT)], den_out.at[c, pl.ds(rb, RPT)])


def kernel(x, edge_index, W1, a_s1, a_d1, b1, W2, a_s2, a_d2, b2):
    src = edge_index[0].reshape(NW, NBLK, 1, NB * K)
    dst = edge_index[1].reshape(NW, NBLK, 1, NB * K)
    z2 = jnp.zeros((NPAD, C), jnp.float32)
    z1 = jnp.zeros((NPAD,), jnp.float32)

    vs1 = W1 @ a_s1.reshape(C)
    vd1 = W1 @ a_d1.reshape(C)
    vs2 = W2 @ a_s2.reshape(C)
    vd2 = W2 @ a_d2.reshape(C)

    h1, as1, ad1 = _mm_logits(x, W1, vs1.reshape(1, C), vd1.reshape(1, C))
    acc1, den1 = _sc_edge(h1, as1, ad1, src, dst, z2, z1)
    h2, as2, ad2 = _norm_mm_logits(
        acc1, den1, b1.reshape(1, C), W2, vs2.reshape(1, C), vd2.reshape(1, C))
    acc2, den2 = _sc_edge(h2, as2, ad2, src, dst, z2, z1)
    out = _norm_out(acc2, den2, b2.reshape(1, C))
    return out
